# Initial kernel scaffold; baseline (speedup 1.0000x reference)
import jax
import jax.numpy as jnp
from jax.experimental import pallas as pl


def kernel(theta0, theta1, obs0, obs1, mapping0, idx1):
    raise NotImplementedError("write your pallas kernel here")


# speedup vs baseline: 1.1343x; 1.1343x over previous
import jax
import jax.numpy as jnp
from jax import lax
from jax.experimental import pallas as pl
from jax.experimental.pallas import tpu as pltpu

_B1 = 64
_B0 = 32
_S1 = 8
_S0 = 8
_STEPS = _S1 + _S0


def _loss_body(idx_ref, theta1_ref, theta0_ref, map_ref, obs0_ref, obs1_ref,
               out_ref, proc0_sc, s1_sc):
    i = pl.program_id(0)

    @pl.when(i == 0)
    def _init():
        s1_sc[...] = jnp.zeros_like(s1_sc)

    @pl.when(i < _S1)
    def _theta1():
        s = jnp.sum(jnp.exp(theta1_ref[...]), axis=2)
        idxb = idx_ref[0, 0, :]
        oh = (lax.broadcasted_iota(jnp.int32, (64, _B1), 0)
              == idxb[None, :]).astype(jnp.float32)
        s1_sc[...] += jnp.dot(oh, s, preferred_element_type=jnp.float32)

    @pl.when(i >= _S1)
    def _theta0():
        j = i - _S1
        s0 = jnp.sum(jnp.exp(theta0_ref[...]), axis=2)
        proc0_sc[pl.ds(j * _B0, _B0), :] = s0

    @pl.when(i == _STEPS - 1)
    def _final():
        p = jnp.dot(map_ref[...], proc0_sc[...],
                    preferred_element_type=jnp.float32)
        d0 = obs0_ref[...] - p
        l0 = jnp.sum(d0 * d0, keepdims=True) * (1.0 / (2048.0 * 256.0))
        d1 = obs1_ref[...] - s1_sc[...]
        l1 = jnp.sum(d1 * d1, keepdims=True) * (1.0 / (64.0 * 128.0))
        out_ref[...] = (0.5 * (l0 + l1)).reshape(1, 1)


def kernel(theta0, theta1, obs0, obs1, mapping0, idx1):
    idx3 = idx1.astype(jnp.int32).reshape(_S1, 1, _B1)
    out = pl.pallas_call(
        _loss_body,
        grid=(_STEPS,),
        in_specs=[
            pl.BlockSpec((1, 1, _B1), lambda i: (jnp.minimum(i, _S1 - 1), 0, 0)),
            pl.BlockSpec((_B1, 128, 128), lambda i: (jnp.minimum(i, _S1 - 1), 0, 0)),
            pl.BlockSpec((_B0, 256, 64), lambda i: (jnp.maximum(i - _S1, 0), 0, 0)),
            pl.BlockSpec((2048, 256), lambda i: (0, 0)),
            pl.BlockSpec((2048, 256), lambda i: (0, 0)),
            pl.BlockSpec((64, 128), lambda i: (0, 0)),
        ],
        out_specs=pl.BlockSpec((1, 1), lambda i: (0, 0)),
        out_shape=jax.ShapeDtypeStruct((1, 1), jnp.float32),
        scratch_shapes=[
            pltpu.VMEM((256, 256), jnp.float32),
            pltpu.VMEM((64, 128), jnp.float32),
        ],
        compiler_params=pltpu.CompilerParams(
            dimension_semantics=("arbitrary",),
        ),
    )(idx3, theta1, theta0, mapping0, obs0, obs1)
    return out[0, 0]


# speedup vs baseline: 2.2987x; 2.0265x over previous
import jax
import jax.numpy as jnp
from jax import lax
from jax.experimental import pallas as pl
from jax.experimental.pallas import tpu as pltpu

_B1 = 64
_B0 = 32
_S1 = 8
_S0 = 8
_STEPS = _S1 + _S0


def _loss_body(idx_ref, theta1_ref, theta0_ref, map_ref, obs0_ref, obs1_ref,
               out_ref, proc0_sc, s1_sc):
    i = pl.program_id(0)

    @pl.when(i == 0)
    def _init():
        s1_sc[...] = jnp.zeros_like(s1_sc)

    @pl.when(i < _S1)
    def _theta1():
        s = jnp.sum(jnp.exp(theta1_ref[...]), axis=2)
        idxb = idx_ref[0, 0, :]
        oh = (lax.broadcasted_iota(jnp.int32, (64, _B1), 0)
              == idxb[None, :]).astype(jnp.float32)
        s1_sc[...] += jnp.dot(oh, s, preferred_element_type=jnp.float32)

    @pl.when(i >= _S1)
    def _theta0():
        j = i - _S1
        s0 = jnp.sum(jnp.exp(theta0_ref[...]), axis=1)
        proc0_sc[pl.ds(j * _B0, _B0), :] = s0

    @pl.when(i == _STEPS - 1)
    def _final():
        p = jnp.dot(map_ref[...], proc0_sc[...],
                    preferred_element_type=jnp.float32)
        d0 = obs0_ref[...] - p
        l0 = jnp.sum(d0 * d0, keepdims=True) * (1.0 / (2048.0 * 256.0))
        d1 = obs1_ref[...] - s1_sc[...]
        l1 = jnp.sum(d1 * d1, keepdims=True) * (1.0 / (64.0 * 128.0))
        out_ref[...] = (0.5 * (l0 + l1)).reshape(1, 1)


def kernel(theta0, theta1, obs0, obs1, mapping0, idx1):
    idx3 = idx1.astype(jnp.int32).reshape(_S1, 1, _B1)
    theta0_t = jnp.transpose(theta0, (0, 2, 1))
    out = pl.pallas_call(
        _loss_body,
        grid=(_STEPS,),
        in_specs=[
            pl.BlockSpec((1, 1, _B1), lambda i: (jnp.minimum(i, _S1 - 1), 0, 0)),
            pl.BlockSpec((_B1, 128, 128), lambda i: (jnp.minimum(i, _S1 - 1), 0, 0)),
            pl.BlockSpec((_B0, 64, 256), lambda i: (jnp.maximum(i - _S1, 0), 0, 0)),
            pl.BlockSpec((2048, 256), lambda i: (0, 0)),
            pl.BlockSpec((2048, 256), lambda i: (0, 0)),
            pl.BlockSpec((64, 128), lambda i: (0, 0)),
        ],
        out_specs=pl.BlockSpec((1, 1), lambda i: (0, 0)),
        out_shape=jax.ShapeDtypeStruct((1, 1), jnp.float32),
        scratch_shapes=[
            pltpu.VMEM((256, 256), jnp.float32),
            pltpu.VMEM((64, 128), jnp.float32),
        ],
        compiler_params=pltpu.CompilerParams(
            dimension_semantics=("arbitrary",),
        ),
    )(idx3, theta1, theta0_t, mapping0, obs0, obs1)
    return out[0, 0]


# speedup vs baseline: 2.4155x; 1.0508x over previous
import jax
import jax.numpy as jnp
from jax import lax
from jax.experimental import pallas as pl
from jax.experimental.pallas import tpu as pltpu

_B0 = 32
_B1 = 64
_S0 = 8
_S1 = 8
_STEPS = _S0 + _S1


def _loss_body(idx_ref, theta0_ref, theta1_ref, map_ref, obs0_ref, obs1_ref,
               out_ref, proc0_sc, s1_sc, l0_sc):
    i = pl.program_id(0)

    @pl.when(i < _S0)
    def _theta0():
        s0 = jnp.sum(jnp.exp(theta0_ref[...]), axis=1)
        proc0_sc[pl.ds(i * _B0, _B0), :] = s0

    @pl.when(i == _S0)
    def _loss0():
        p = jnp.dot(map_ref[...], proc0_sc[...],
                    preferred_element_type=jnp.float32)
        d0 = obs0_ref[...] - p
        l0_sc[0] = jnp.sum(d0 * d0) * (1.0 / (2048.0 * 256.0))
        s1_sc[...] = jnp.zeros_like(s1_sc)

    @pl.when(i >= _S0)
    def _theta1():
        s = jnp.sum(jnp.exp(theta1_ref[...]), axis=2)
        idxb = idx_ref[0, 0, :]
        oh = (lax.broadcasted_iota(jnp.int32, (64, _B1), 0)
              == idxb[None, :]).astype(jnp.float32)
        s1_sc[...] += jnp.dot(oh, s, preferred_element_type=jnp.float32)

    @pl.when(i == _STEPS - 1)
    def _final():
        d1 = obs1_ref[...] - s1_sc[...]
        l1 = jnp.sum(d1 * d1, keepdims=True) * (1.0 / (64.0 * 128.0))
        out_ref[...] = (0.5 * (l0_sc[0] + l1)).reshape(1, 1)


def kernel(theta0, theta1, obs0, obs1, mapping0, idx1):
    idx3 = idx1.astype(jnp.int32).reshape(_S1, 1, _B1)
    theta0_t = jnp.transpose(theta0, (0, 2, 1))
    out = pl.pallas_call(
        _loss_body,
        grid=(_STEPS,),
        in_specs=[
            pl.BlockSpec((1, 1, _B1), lambda i: (jnp.maximum(i - _S0, 0), 0, 0)),
            pl.BlockSpec((_B0, 64, 256), lambda i: (jnp.minimum(i, _S0 - 1), 0, 0)),
            pl.BlockSpec((_B1, 128, 128), lambda i: (jnp.maximum(i - _S0, 0), 0, 0)),
            pl.BlockSpec((2048, 256), lambda i: (0, 0)),
            pl.BlockSpec((2048, 256), lambda i: (0, 0)),
            pl.BlockSpec((64, 128), lambda i: (0, 0)),
        ],
        out_specs=pl.BlockSpec((1, 1), lambda i: (0, 0)),
        out_shape=jax.ShapeDtypeStruct((1, 1), jnp.float32),
        scratch_shapes=[
            pltpu.VMEM((256, 256), jnp.float32),
            pltpu.VMEM((64, 128), jnp.float32),
            pltpu.SMEM((1,), jnp.float32),
        ],
        compiler_params=pltpu.CompilerParams(
            dimension_semantics=("arbitrary",),
        ),
    )(idx3, theta0_t, theta1, mapping0, obs0, obs1)
    return out[0, 0]


# speedup vs baseline: 2.4279x; 1.0051x over previous
import jax
import jax.numpy as jnp
from jax import lax
from jax.experimental import pallas as pl
from jax.experimental.pallas import tpu as pltpu

_B0 = 16
_B1 = 32
_S0 = 8
_S1 = 8
_STEPS = _S0 + _S1


def _loss_body(idxa_ref, idxb_ref, t0a_ref, t0b_ref, t1a_ref, t1b_ref,
               map_ref, obs0_ref, obs1_ref,
               out_ref, proc0_sc, s1_sc, l0_sc):
    i = pl.program_id(0)

    @pl.when(i < _S0)
    def _theta0():
        s0a = jnp.sum(jnp.exp(t0a_ref[...]), axis=1)
        s0b = jnp.sum(jnp.exp(t0b_ref[...]), axis=1)
        proc0_sc[pl.ds(i * 2 * _B0, _B0), :] = s0a
        proc0_sc[pl.ds(i * 2 * _B0 + _B0, _B0), :] = s0b

    @pl.when(i == _S0)
    def _loss0():
        p = jnp.dot(map_ref[...], proc0_sc[...],
                    preferred_element_type=jnp.float32)
        d0 = obs0_ref[...] - p
        l0_sc[0] = jnp.sum(d0 * d0) * (1.0 / (2048.0 * 256.0))
        s1_sc[...] = jnp.zeros_like(s1_sc)

    @pl.when(i >= _S0)
    def _theta1():
        sa = jnp.sum(jnp.exp(t1a_ref[...]), axis=2)
        sb = jnp.sum(jnp.exp(t1b_ref[...]), axis=2)
        ia = idxa_ref[0, 0, :]
        ib = idxb_ref[0, 0, :]
        rows = lax.broadcasted_iota(jnp.int32, (64, _B1), 0)
        oha = (rows == ia[None, :]).astype(jnp.float32)
        ohb = (rows == ib[None, :]).astype(jnp.float32)
        s1_sc[...] += (jnp.dot(oha, sa, preferred_element_type=jnp.float32)
                       + jnp.dot(ohb, sb, preferred_element_type=jnp.float32))

    @pl.when(i == _STEPS - 1)
    def _final():
        d1 = obs1_ref[...] - s1_sc[...]
        l1 = jnp.sum(d1 * d1, keepdims=True) * (1.0 / (64.0 * 128.0))
        out_ref[...] = (0.5 * (l0_sc[0] + l1)).reshape(1, 1)


def kernel(theta0, theta1, obs0, obs1, mapping0, idx1):
    idx3 = idx1.astype(jnp.int32).reshape(2 * _S1, 1, _B1)
    theta0_t = jnp.transpose(theta0, (0, 2, 1))

    def _ma(i):
        return jnp.maximum(2 * (i - _S0), 0)

    out = pl.pallas_call(
        _loss_body,
        grid=(_STEPS,),
        in_specs=[
            pl.BlockSpec((1, 1, _B1), lambda i: (_ma(i), 0, 0)),
            pl.BlockSpec((1, 1, _B1), lambda i: (_ma(i) + 1, 0, 0)),
            pl.BlockSpec((_B0, 64, 256), lambda i: (jnp.minimum(2 * i, 14), 0, 0)),
            pl.BlockSpec((_B0, 64, 256), lambda i: (jnp.minimum(2 * i + 1, 15), 0, 0)),
            pl.BlockSpec((_B1, 128, 128), lambda i: (_ma(i), 0, 0)),
            pl.BlockSpec((_B1, 128, 128), lambda i: (_ma(i) + 1, 0, 0)),
            pl.BlockSpec((2048, 256), lambda i: (0, 0)),
            pl.BlockSpec((2048, 256), lambda i: (0, 0)),
            pl.BlockSpec((64, 128), lambda i: (0, 0)),
        ],
        out_specs=pl.BlockSpec((1, 1), lambda i: (0, 0)),
        out_shape=jax.ShapeDtypeStruct((1, 1), jnp.float32),
        scratch_shapes=[
            pltpu.VMEM((256, 256), jnp.float32),
            pltpu.VMEM((64, 128), jnp.float32),
            pltpu.SMEM((1,), jnp.float32),
        ],
        compiler_params=pltpu.CompilerParams(
            dimension_semantics=("arbitrary",),
        ),
    )(idx3, idx3, theta0_t, theta0_t, theta1, theta1, mapping0, obs0, obs1)
    return out[0, 0]
